# baseline (speedup 1.0000x reference)
import functools

import jax
import jax.numpy as jnp
from jax.experimental import pallas as pl
from jax.experimental.pallas import tpu as pltpu

N_CELLS = 10000
N_GENES = 2000
BC = 2000
NB = N_CELLS // BC
BCT = 1000
NBT = N_CELLS // BCT
BG = 1024


def _layer_kernel(a_ref, c_ref, g_ref,
                  csw_ref, csb_ref, cnw_ref, cnb_ref,
                  gsw_ref, gsb_ref, gnw_ref, gnb_ref,
                  c_out_ref, g_out_ref, gn_acc):
    i = pl.program_id(0)
    a = a_ref[...]
    c = c_ref[...]

    cn = jnp.dot(a, g_ref[...], preferred_element_type=jnp.float32)
    c_new = (jnp.dot(c, csw_ref[...].T, preferred_element_type=jnp.float32)
             + csb_ref[...]
             + jnp.dot(cn, cnw_ref[...].T, preferred_element_type=jnp.float32)
             + cnb_ref[...])
    c_out_ref[...] = jnp.maximum(c_new, 0.0)

    contrib = jax.lax.dot_general(
        a, c, (((0,), (0,)), ((), ())), preferred_element_type=jnp.float32)

    @pl.when(i == 0)
    def _():
        gn_acc[...] = contrib

    @pl.when(i > 0)
    def _():
        gn_acc[...] += contrib

    @pl.when(i == NB - 1)
    def _():
        g_new = (jnp.dot(g_ref[...], gsw_ref[...].T,
                         preferred_element_type=jnp.float32)
                 + gsb_ref[...]
                 + jnp.dot(gn_acc[...], gnw_ref[...].T,
                           preferred_element_type=jnp.float32)
                 + gnb_ref[...])
        g_out_ref[...] = jnp.maximum(g_new, 0.0)


def _run_layer(adj, c, g, csw, csb, cnw, cnb, gsw, gsb, gnw, gnb):
    cf = c.shape[1]
    gf = g.shape[1]
    h = csw.shape[0]
    full = lambda shp: pl.BlockSpec(shp, lambda i: (0, 0))
    return pl.pallas_call(
        _layer_kernel,
        grid=(NB,),
        in_specs=[
            pl.BlockSpec((BC, N_GENES), lambda i: (i, 0)),
            pl.BlockSpec((BC, cf), lambda i: (i, 0)),
            full((N_GENES, gf)),
            full(csw.shape), full(csb.shape),
            full(cnw.shape), full(cnb.shape),
            full(gsw.shape), full(gsb.shape),
            full(gnw.shape), full(gnb.shape),
        ],
        out_specs=[
            pl.BlockSpec((BC, h), lambda i: (i, 0)),
            full((N_GENES, h)),
        ],
        out_shape=[
            jax.ShapeDtypeStruct((N_CELLS, h), jnp.float32),
            jax.ShapeDtypeStruct((N_GENES, h), jnp.float32),
        ],
        scratch_shapes=[pltpu.VMEM((N_GENES, h), jnp.float32)],
    )(adj, c, g, csw, csb, cnw, cnb, gsw, gsb, gnw, gnb)


def _tail_kernel(c_ref, g_ref, clw_ref, clb_ref, glw_ref, glb_ref,
                 cdw_ref, cdb_ref, gdw_ref, gdb_ref, cen_ref,
                 zc_ref, zg_ref, crec_ref, grec_ref, q_ref, zg_scratch):
    i = pl.program_id(0)

    zc = (jnp.dot(c_ref[...], clw_ref[...].T,
                  preferred_element_type=jnp.float32) + clb_ref[...])
    zc_ref[...] = zc
    crec_ref[...] = (jnp.dot(zc, cdw_ref[...].T,
                             preferred_element_type=jnp.float32)
                     + cdb_ref[...])

    @pl.when(i == 0)
    def _():
        zg = (jnp.dot(g_ref[...], glw_ref[...].T,
                      preferred_element_type=jnp.float32) + glb_ref[...])
        zg_scratch[...] = zg
        zg_ref[...] = zg

    grec_ref[...] = (jnp.dot(zg_scratch[...], gdw_ref[...].T,
                             preferred_element_type=jnp.float32)
                     + gdb_ref[...])

    cen = cen_ref[...]
    k = cen.shape[0]
    cols = []
    for j in range(k):
        diff = zc - cen[j, :][None, :]
        cols.append(jnp.sum(diff * diff, axis=1, keepdims=True))
    d = jnp.concatenate(cols, axis=1)
    num = 1.0 / (1.0 + d)
    q_ref[...] = num / jnp.sum(num, axis=1, keepdims=True)


def _run_tail(c, g, clw, clb, glw, glb, cdw, cdb, gdw, gdb, centers):
    h = c.shape[1]
    lat = clw.shape[0]
    k = centers.shape[0]
    full = lambda shp: pl.BlockSpec(shp, lambda i: (0, 0))
    return pl.pallas_call(
        _tail_kernel,
        grid=(NBT,),
        in_specs=[
            pl.BlockSpec((BCT, h), lambda i: (i, 0)),
            full((N_GENES, h)),
            full(clw.shape), full(clb.shape),
            full(glw.shape), full(glb.shape),
            full(cdw.shape), full(cdb.shape),
            pl.BlockSpec((BG, lat), lambda i: (i, 0)),
            pl.BlockSpec((1, BG), lambda i: (0, i)),
            full(centers.shape),
        ],
        out_specs=[
            pl.BlockSpec((BCT, lat), lambda i: (i, 0)),
            full((N_GENES, lat)),
            pl.BlockSpec((BCT, N_GENES), lambda i: (i, 0)),
            pl.BlockSpec((N_GENES, BG), lambda i: (0, i)),
            pl.BlockSpec((BCT, k), lambda i: (i, 0)),
        ],
        out_shape=[
            jax.ShapeDtypeStruct((N_CELLS, lat), jnp.float32),
            jax.ShapeDtypeStruct((N_GENES, lat), jnp.float32),
            jax.ShapeDtypeStruct((N_CELLS, N_GENES), jnp.float32),
            jax.ShapeDtypeStruct((N_GENES, N_CELLS), jnp.float32),
            jax.ShapeDtypeStruct((N_CELLS, k), jnp.float32),
        ],
        scratch_shapes=[pltpu.VMEM((N_GENES, lat), jnp.float32)],
    )(c, g, clw, clb, glw, glb, cdw, cdb, gdw, gdb, centers)


@jax.jit
def kernel(cell_x, gene_x, adjacency,
           l0_cs_w, l0_cs_b, l0_cn_w, l0_cn_b, l0_gs_w, l0_gs_b, l0_gn_w, l0_gn_b,
           l1_cs_w, l1_cs_b, l1_cn_w, l1_cn_b, l1_gs_w, l1_gs_b, l1_gn_w, l1_gn_b,
           cl_w, cl_b, gl_w, gl_b, cd_w, cd_b, gd_w, gd_b, centers):
    r2 = lambda b: b.reshape(1, -1)
    c, g = cell_x, gene_x
    c, g = _run_layer(adjacency, c, g,
                      l0_cs_w, r2(l0_cs_b), l0_cn_w, r2(l0_cn_b),
                      l0_gs_w, r2(l0_gs_b), l0_gn_w, r2(l0_gn_b))
    c, g = _run_layer(adjacency, c, g,
                      l1_cs_w, r2(l1_cs_b), l1_cn_w, r2(l1_cn_b),
                      l1_gs_w, r2(l1_gs_b), l1_gn_w, r2(l1_gn_b))
    z_cells, z_genes, cell_recon, gene_recon, q = _run_tail(
        c, g, cl_w, r2(cl_b), gl_w, r2(gl_b),
        cd_w, r2(cd_b), gd_w, r2(gd_b), centers)
    return (z_cells, z_genes, cell_recon, gene_recon, q)
